# baseline (speedup 1.0000x reference)
import jax
import jax.numpy as jnp
from jax.experimental import pallas as pl

_H = 256
_E = 8
_TILE = 512


def _moe_dense_kernel(x_ref, wg_ref, w2_ref, be_ref, o_ref):
    x = x_ref[...]
    logits = jnp.dot(x, wg_ref[...], preferred_element_type=jnp.float32)
    m = jnp.max(logits, axis=-1, keepdims=True)
    denom = jnp.sum(jnp.exp(logits - m), axis=-1, keepdims=True)
    gate = 1.0 / denom
    idx = jnp.argmax(logits, axis=-1)
    onehot = (jax.lax.broadcasted_iota(jnp.int32, (_TILE, _E), 1)
              == idx[:, None]).astype(jnp.float32)
    r = jnp.dot(x.astype(jnp.bfloat16), w2_ref[...],
                preferred_element_type=jnp.float32)
    r3 = r.reshape(_TILE, _E, _H)
    sel = jnp.sum(r3 * onehot[:, :, None], axis=1)
    bsel = jnp.dot(onehot, be_ref[...], preferred_element_type=jnp.float32)
    o_ref[...] = (sel + bsel) * gate


def kernel(x, Wg, We, be):
    B, S, H = x.shape
    xt = x.reshape(-1, H)
    T = xt.shape[0]
    W2 = We.transpose(1, 0, 2).reshape(H, _E * H).astype(jnp.bfloat16)
    out = pl.pallas_call(
        _moe_dense_kernel,
        grid=(T // _TILE,),
        in_specs=[
            pl.BlockSpec((_TILE, H), lambda i: (i, 0)),
            pl.BlockSpec((H, _E), lambda i: (0, 0)),
            pl.BlockSpec((H, _E * H), lambda i: (0, 0)),
            pl.BlockSpec((_E, H), lambda i: (0, 0)),
        ],
        out_specs=pl.BlockSpec((_TILE, H), lambda i: (i, 0)),
        out_shape=jax.ShapeDtypeStruct((T, H), jnp.float32),
    )(xt, Wg, W2, be)
    return out.reshape(B, S, H)


# speedup vs baseline: 1.8442x; 1.8442x over previous
import jax
import jax.numpy as jnp
from jax.experimental import pallas as pl

_H = 256
_E = 8
_TILE = 512


def _moe_dense_kernel(x_ref, wg_ref, ws_ref, be_ref, o_ref):
    x = x_ref[...]
    logits = jnp.dot(x, wg_ref[...], preferred_element_type=jnp.float32)
    m = jnp.max(logits, axis=-1, keepdims=True)
    denom = jnp.sum(jnp.exp(logits - m), axis=-1, keepdims=True)
    gate = 1.0 / denom
    idx = jnp.argmax(logits, axis=-1)
    onehot = (jax.lax.broadcasted_iota(jnp.int32, (_TILE, _E), 1)
              == idx[:, None])
    og = jnp.where(onehot, gate, 0.0)
    x8 = jnp.concatenate(
        [(x * og[:, e:e + 1]).astype(jnp.bfloat16) for e in range(_E)],
        axis=1)
    acc = jnp.dot(og, be_ref[...], preferred_element_type=jnp.float32)
    acc = acc + jnp.dot(x8, ws_ref[...], preferred_element_type=jnp.float32)
    o_ref[...] = acc


def kernel(x, Wg, We, be):
    B, S, H = x.shape
    xt = x.reshape(-1, H)
    T = xt.shape[0]
    Ws = We.reshape(_E * H, H).astype(jnp.bfloat16)
    out = pl.pallas_call(
        _moe_dense_kernel,
        grid=(T // _TILE,),
        in_specs=[
            pl.BlockSpec((_TILE, H), lambda i: (i, 0)),
            pl.BlockSpec((H, _E), lambda i: (0, 0)),
            pl.BlockSpec((_E * H, H), lambda i: (0, 0)),
            pl.BlockSpec((_E, H), lambda i: (0, 0)),
        ],
        out_specs=pl.BlockSpec((_TILE, H), lambda i: (i, 0)),
        out_shape=jax.ShapeDtypeStruct((T, H), jnp.float32),
    )(xt, Wg, Ws, be)
    return out.reshape(B, S, H)
